# baseline (device time: 37757 ns/iter reference)
import jax
import jax.numpy as jnp
from jax import lax
from jax.experimental import pallas as pl
from jax.experimental.pallas import tpu as pltpu

N_DEV = 32
SPLIT = 16


def kernel(x):
    m, n = x.shape
    c = m // N_DEV

    def body(x_ref, out_ref, rs_recv, rs_send_sems, rs_recv_sems,
             ag_send_sems, ag_recv_sems):
        my = lax.axis_index("i")

        barrier_sem = pltpu.get_barrier_semaphore()
        for k in range(1, N_DEV):
            pl.semaphore_signal(
                barrier_sem, inc=1,
                device_id=((my + k) % N_DEV,),
                device_id_type=pl.DeviceIdType.MESH,
            )
        pl.semaphore_wait(barrier_sem, N_DEV - 1)

        def rs_desc(k):
            p = (my + k) % N_DEV
            return pltpu.make_async_remote_copy(
                src_ref=x_ref.at[pl.ds(p * c, c), :],
                dst_ref=rs_recv.at[N_DEV - k],
                send_sem=rs_send_sems.at[k],
                recv_sem=rs_recv_sems.at[N_DEV - k],
                device_id=(p,),
                device_id_type=pl.DeviceIdType.MESH,
            )

        def rs_wait_desc(k):
            return pltpu.make_async_remote_copy(
                src_ref=x_ref.at[pl.ds(0, c), :],
                dst_ref=rs_recv.at[N_DEV - k],
                send_sem=rs_send_sems.at[k],
                recv_sem=rs_recv_sems.at[N_DEV - k],
                device_id=((my + k) % N_DEV,),
                device_id_type=pl.DeviceIdType.MESH,
            )

        def ag_desc(k):
            p = (my + k) % N_DEV
            return pltpu.make_async_remote_copy(
                src_ref=out_ref.at[pl.ds(my * c, c), :],
                dst_ref=out_ref.at[pl.ds(my * c, c), :],
                send_sem=ag_send_sems.at[k],
                recv_sem=ag_recv_sems.at[N_DEV - k],
                device_id=(p,),
                device_id_type=pl.DeviceIdType.MESH,
            )

        def ag_wait_desc(k):
            p = (my + k) % N_DEV
            return pltpu.make_async_remote_copy(
                src_ref=out_ref.at[pl.ds(my * c, c), :],
                dst_ref=out_ref.at[pl.ds(p * c, c), :],
                send_sem=ag_send_sems.at[k],
                recv_sem=ag_recv_sems.at[N_DEV - k],
                device_id=(p,),
                device_id_type=pl.DeviceIdType.MESH,
            )

        for k in range(1, N_DEV):
            rs_desc(k).start()
        rs_recv[pl.ds(0, 1), :, :] = x_ref[pl.ds(my * c, c), :][None]

        for k in range(1, SPLIT + 1):
            rs_wait_desc(k).wait_recv()
        partial = jnp.sum(rs_recv[N_DEV - SPLIT:, :, :], axis=0)
        for k in range(SPLIT + 1, N_DEV):
            rs_wait_desc(k).wait_recv()
        reduced = partial + jnp.sum(rs_recv[:N_DEV - SPLIT, :, :], axis=0)
        out_ref[pl.ds(my * c, c), :] = reduced

        for k in range(1, N_DEV):
            ag_desc(k).start()
        for k in range(1, N_DEV):
            ag_wait_desc(k).wait_recv()

        for k in range(1, N_DEV):
            rs_wait_desc(k).wait_send()
            ag_wait_desc(k).wait_send()

    return pl.pallas_call(
        body,
        out_shape=jax.ShapeDtypeStruct((m, n), x.dtype),
        in_specs=[pl.BlockSpec(memory_space=pltpu.VMEM)],
        out_specs=pl.BlockSpec(memory_space=pltpu.VMEM),
        scratch_shapes=[
            pltpu.VMEM((N_DEV, c, n), x.dtype),
            pltpu.SemaphoreType.DMA((N_DEV,)),
            pltpu.SemaphoreType.DMA((N_DEV,)),
            pltpu.SemaphoreType.DMA((N_DEV,)),
            pltpu.SemaphoreType.DMA((N_DEV,)),
        ],
        compiler_params=pltpu.CompilerParams(collective_id=0),
    )(x)
